# Initial kernel scaffold; baseline (speedup 1.0000x reference)
import jax
import jax.numpy as jnp
from jax.experimental import pallas as pl


def kernel(x, table):
    raise NotImplementedError("write your pallas kernel here")


# speedup vs baseline: 3.5665x; 3.5665x over previous
import functools

import jax
import jax.numpy as jnp
from jax import lax
from jax.experimental import pallas as pl
from jax.experimental.pallas import tpu as pltpu
from jax.experimental.pallas import tpu_sc as plsc

_HIDDEN = 1024
_VOCAB = 2


@functools.lru_cache(maxsize=None)
def _build_sc_embed(n_rows: int, vocab: int, hidden: int):
    info = plsc.get_sparse_core_info()
    nc, ns = info.num_cores, info.num_subcores
    nw = nc * ns
    assert n_rows % nw == 0
    rows_per_w = n_rows // nw

    mesh = plsc.VectorSubcoreMesh(core_axis_name="c", subcore_axis_name="s")

    @functools.partial(
        pl.kernel,
        out_type=jax.ShapeDtypeStruct((n_rows, hidden), jnp.float32),
        mesh=mesh,
        scratch_types=[
            pltpu.VMEM((vocab, hidden), jnp.float32),
            pltpu.VMEM((rows_per_w,), jnp.int32),
            pltpu.SemaphoreType.DMA,
        ],
    )
    def embed(x_hbm, table_hbm, out_hbm, table_v, idx_v, sem):
        wid = lax.axis_index("s") * nc + lax.axis_index("c")
        base = wid * rows_per_w
        pltpu.sync_copy(table_hbm, table_v)
        pltpu.sync_copy(x_hbm.at[pl.ds(base, rows_per_w)], idx_v)

        lanes = 16

        def body(g, carry):
            row0 = g * lanes
            xv = idx_v[pl.ds(row0, lanes)]
            for j in range(lanes):
                pltpu.make_async_copy(
                    table_v.at[xv[j]], out_hbm.at[base + row0 + j], sem
                ).start()
            return carry

        lax.fori_loop(0, rows_per_w // lanes, body, 0)
        pltpu.make_async_copy(
            out_hbm.at[pl.ds(base, rows_per_w)],
            out_hbm.at[pl.ds(base, rows_per_w)],
            sem,
        ).wait()

    return embed


def kernel(x, table):
    b, s = x.shape
    n = b * s
    xf = x.reshape(n).astype(jnp.int32)
    out_flat = _build_sc_embed(n, table.shape[0], table.shape[1])(xf, table)
    return out_flat.reshape(b, s, table.shape[1])
